# baseline (speedup 1.0000x reference)
import functools

import jax
import jax.numpy as jnp
from jax.experimental import pallas as pl
from jax.experimental.pallas import tpu as pltpu

N, F, H, C, Q = 10000, 128, 16, 16, 2048

BM1 = 400
G = 8


def _matmul_small_kernel(a_ref, b_ref, o_ref):
    o_ref[...] = jnp.dot(a_ref[...], b_ref[...],
                         preferred_element_type=jnp.float32)


def _layer1_kernel(adj_ref, xw_ref, h_ref):
    acc = jnp.dot(adj_ref[...], xw_ref[...],
                  preferred_element_type=jnp.float32)
    h_ref[...] = jnp.maximum(acc, 0.0)


def _layer2_kernel(idx_ref, *refs):
    adj_rows = refs[:G]
    h_ref, w1_ref, out_ref = refs[G], refs[G + 1], refs[G + 2]
    rows = jnp.concatenate([r[0] for r in adj_rows], axis=0)
    hrow = jnp.dot(rows, h_ref[...], preferred_element_type=jnp.float32)
    out_ref[...] = jnp.dot(hrow, w1_ref[...],
                           preferred_element_type=jnp.float32)


@functools.partial(jax.jit, static_argnames=())
def kernel(x, adj, idx, W0, W1):
    xw = pl.pallas_call(
        _matmul_small_kernel,
        out_shape=jax.ShapeDtypeStruct((N, H), jnp.float32),
    )(x, W0)

    nm = N // BM1
    h = pl.pallas_call(
        _layer1_kernel,
        grid=(nm,),
        in_specs=[
            pl.BlockSpec((BM1, N), lambda m: (m, 0)),
            pl.BlockSpec((N, H), lambda m: (0, 0)),
        ],
        out_specs=pl.BlockSpec((BM1, H), lambda m: (m, 0)),
        out_shape=jax.ShapeDtypeStruct((N, H), jnp.float32),
    )(adj, xw)

    adj3 = adj.reshape(N, 1, N)

    def _row_map(j):
        def index_map(q, idx_ref):
            return (idx_ref[q * G + j], 0, 0)
        return index_map

    grid_spec = pltpu.PrefetchScalarGridSpec(
        num_scalar_prefetch=1,
        grid=(Q // G,),
        in_specs=(
            [pl.BlockSpec((1, 1, N), _row_map(j)) for j in range(G)]
            + [
                pl.BlockSpec((N, H), lambda q, idx_ref: (0, 0)),
                pl.BlockSpec((H, C), lambda q, idx_ref: (0, 0)),
            ]
        ),
        out_specs=pl.BlockSpec((G, C), lambda q, idx_ref: (q, 0)),
    )
    out = pl.pallas_call(
        _layer2_kernel,
        grid_spec=grid_spec,
        out_shape=jax.ShapeDtypeStruct((Q, C), jnp.float32),
    )(idx, *([adj3] * G), h, W1)
    return out


# speedup vs baseline: 1.0607x; 1.0607x over previous
import functools

import jax
import jax.numpy as jnp
from jax import lax
from jax.experimental import pallas as pl
from jax.experimental.pallas import tpu as pltpu
from jax.experimental.pallas import tpu_sc as plsc

N, F, H, C, Q = 10000, 128, 16, 16, 2048

BM1 = 400
SUB = 5
CW = N // SUB
BR2 = 256


def _matmul_small_kernel(a_ref, b_ref, o_ref):
    o_ref[...] = jnp.dot(a_ref[...], b_ref[...],
                         preferred_element_type=jnp.float32)


def _layer1_kernel(adj_ref, xw_ref, h_ref):
    acc = jnp.dot(adj_ref[...], xw_ref[...],
                  preferred_element_type=jnp.float32)
    h_ref[...] = jnp.maximum(acc, 0.0)


def _layer2_kernel(g_ref, h_ref, w1_ref, out_ref):
    hr = jnp.dot(g_ref[...], h_ref[...], preferred_element_type=jnp.float32)
    out_ref[...] = jnp.dot(hr, w1_ref[...],
                           preferred_element_type=jnp.float32)


def _sc_gather(adj16, idx):
    info = plsc.get_sparse_core_info()
    nw = info.num_cores * info.num_subcores
    rpw = Q // nw
    nchunk = rpw // 16
    ntask = nchunk * SUB
    mesh = plsc.VectorSubcoreMesh(core_axis_name="c", subcore_axis_name="s")

    @functools.partial(
        pl.kernel,
        mesh=mesh,
        compiler_params=pltpu.CompilerParams(use_tc_tiling_on_sc=False),
        out_type=jax.ShapeDtypeStruct((Q * SUB, CW), jnp.float32),
        scratch_types=[
            pltpu.VMEM((rpw,), jnp.int32),
            pltpu.VMEM((2, 16, CW), jnp.float32),
            pltpu.SemaphoreType.DMA((2,)),
            pltpu.SemaphoreType.DMA((2,)),
        ],
    )
    def k(adj_hbm, idx_hbm, out_hbm, idx_v, buf, gsem, ssem):
        wid = lax.axis_index("s") * info.num_cores + lax.axis_index("c")
        base = wid * rpw
        pltpu.sync_copy(idx_hbm.at[pl.ds(base, rpw)], idx_v)
        iota16 = lax.broadcasted_iota(jnp.int32, (16,), 0)

        def src_of(t):
            c, j = divmod(t, SUB)
            chunk = idx_v[pl.ds(c * 16, 16)]
            return chunk * SUB + j

        def dst_of(t):
            c, j = divmod(t, SUB)
            return (base + c * 16 + iota16) * SUB + j

        def gather(t, slot):
            return pltpu.make_async_copy(adj_hbm.at[src_of(t)],
                                         buf.at[slot], gsem.at[slot])

        def scatter(t, slot):
            return pltpu.make_async_copy(buf.at[slot],
                                         out_hbm.at[dst_of(t)],
                                         ssem.at[slot])

        for t in range(ntask):
            slot = t % 2
            if t >= 2:
                scatter(t - 2, slot).wait()
            gather(t, slot).start()
            if t >= 1:
                gather(t - 1, 1 - slot).wait()
                scatter(t - 1, 1 - slot).start()
        last = ntask - 1
        gather(last, last % 2).wait()
        scatter(last, last % 2).start()
        scatter(last - 1, (last - 1) % 2).wait()
        scatter(last, last % 2).wait()

    return k(adj16, idx)


@functools.partial(jax.jit, static_argnames=())
def kernel(x, adj, idx, W0, W1):
    adj16 = adj.reshape(N * SUB, CW)
    adjg = _sc_gather(adj16, idx).reshape(Q, N)

    xw = pl.pallas_call(
        _matmul_small_kernel,
        out_shape=jax.ShapeDtypeStruct((N, H), jnp.float32),
    )(x, W0)

    nm = N // BM1
    h = pl.pallas_call(
        _layer1_kernel,
        grid=(nm,),
        in_specs=[
            pl.BlockSpec((BM1, N), lambda m: (m, 0)),
            pl.BlockSpec((N, H), lambda m: (0, 0)),
        ],
        out_specs=pl.BlockSpec((BM1, H), lambda m: (m, 0)),
        out_shape=jax.ShapeDtypeStruct((N, H), jnp.float32),
    )(adj, xw)

    out = pl.pallas_call(
        _layer2_kernel,
        grid=(Q // BR2,),
        in_specs=[
            pl.BlockSpec((BR2, N), lambda b: (b, 0)),
            pl.BlockSpec((N, H), lambda b: (0, 0)),
            pl.BlockSpec((H, C), lambda b: (0, 0)),
        ],
        out_specs=pl.BlockSpec((BR2, C), lambda b: (b, 0)),
        out_shape=jax.ShapeDtypeStruct((Q, C), jnp.float32),
    )(adjg, h, W1)
    return out


# speedup vs baseline: 3.9355x; 3.7102x over previous
import functools

import jax
import jax.numpy as jnp
from jax import lax
from jax.experimental import pallas as pl
from jax.experimental.pallas import tpu as pltpu

N, F, H, C, Q = 10000, 128, 16, 16, 2048

BM1 = 400
BR = 64
NB = Q // BR


def _matmul_small_kernel(a_ref, b_ref, o_ref):
    o_ref[...] = jnp.dot(a_ref[...], b_ref[...],
                         preferred_element_type=jnp.float32)


def _layer1_kernel(adj_ref, xw_ref, h_ref):
    acc = jnp.dot(adj_ref[...], xw_ref[...],
                  preferred_element_type=jnp.float32)
    h_ref[...] = jnp.maximum(acc, 0.0)


def _layer2_kernel(idx_ref, adj_hbm, h_ref, w1_ref, out_ref, buf, sem):
    b = pl.program_id(0)

    def start_batch(batch, slot):
        base = batch * BR
        for j in range(BR):
            pltpu.make_async_copy(
                adj_hbm.at[idx_ref[base + j]],
                buf.at[slot, j],
                sem.at[slot],
            ).start()

    def wait_batch(batch, slot):
        pltpu.make_async_copy(
            adj_hbm.at[pl.ds(0, BR)],
            buf.at[slot],
            sem.at[slot],
        ).wait()

    slot = lax.rem(b, 2)

    @pl.when(b == 0)
    def _():
        start_batch(0, 0)

    @pl.when(b + 1 < NB)
    def _():
        start_batch(b + 1, 1 - slot)

    wait_batch(b, slot)
    rows = buf[slot]
    hr = jnp.dot(rows, h_ref[...], preferred_element_type=jnp.float32)
    out_ref[...] = jnp.dot(hr, w1_ref[...],
                           preferred_element_type=jnp.float32)


@functools.partial(jax.jit, static_argnames=())
def kernel(x, adj, idx, W0, W1):
    xw = pl.pallas_call(
        _matmul_small_kernel,
        out_shape=jax.ShapeDtypeStruct((N, H), jnp.float32),
    )(x, W0)

    nm = N // BM1
    h = pl.pallas_call(
        _layer1_kernel,
        grid=(nm,),
        in_specs=[
            pl.BlockSpec((BM1, N), lambda m: (m, 0)),
            pl.BlockSpec((N, H), lambda m: (0, 0)),
        ],
        out_specs=pl.BlockSpec((BM1, H), lambda m: (m, 0)),
        out_shape=jax.ShapeDtypeStruct((N, H), jnp.float32),
    )(adj, xw)

    grid_spec = pltpu.PrefetchScalarGridSpec(
        num_scalar_prefetch=1,
        grid=(NB,),
        in_specs=[
            pl.BlockSpec(memory_space=pl.ANY),
            pl.BlockSpec((N, H), lambda b, idx_ref: (0, 0)),
            pl.BlockSpec((H, C), lambda b, idx_ref: (0, 0)),
        ],
        out_specs=pl.BlockSpec((BR, C), lambda b, idx_ref: (b, 0)),
        scratch_shapes=[
            pltpu.VMEM((2, BR, N), jnp.float32),
            pltpu.SemaphoreType.DMA((2,)),
        ],
    )
    out = pl.pallas_call(
        _layer2_kernel,
        grid_spec=grid_spec,
        out_shape=jax.ShapeDtypeStruct((Q, C), jnp.float32),
    )(idx, adj, h, W1)
    return out


# speedup vs baseline: 4.1562x; 1.0561x over previous
import functools

import jax
import jax.numpy as jnp
from jax import lax
from jax.experimental import pallas as pl
from jax.experimental.pallas import tpu as pltpu

N, F, H, C, Q = 10000, 128, 16, 16, 2048

BM1 = 400
BR = 128
NB = Q // BR


def _matmul_small_kernel(a_ref, b_ref, o_ref):
    o_ref[...] = jnp.dot(a_ref[...], b_ref[...],
                         preferred_element_type=jnp.float32)


def _layer1_kernel(adj_ref, xw_ref, h_ref):
    acc = jnp.dot(adj_ref[...], xw_ref[...],
                  preferred_element_type=jnp.float32)
    h_ref[...] = jnp.maximum(acc, 0.0)


def _layer2_kernel(idx_ref, adj_hbm, h_ref, w1_ref, out_ref, buf, sem):
    b = pl.program_id(0)

    def start_batch(batch, slot):
        base = batch * BR
        for j in range(BR):
            pltpu.make_async_copy(
                adj_hbm.at[idx_ref[base + j]],
                buf.at[slot, j],
                sem.at[slot],
            ).start()

    def wait_batch(batch, slot):
        pltpu.make_async_copy(
            adj_hbm.at[pl.ds(0, BR)],
            buf.at[slot],
            sem.at[slot],
        ).wait()

    slot = lax.rem(b, 2)

    @pl.when(b == 0)
    def _():
        start_batch(0, 0)

    @pl.when(b + 1 < NB)
    def _():
        start_batch(b + 1, 1 - slot)

    wait_batch(b, slot)
    rows = buf[slot]
    hr = jnp.dot(rows, h_ref[...], preferred_element_type=jnp.float32)
    out_ref[...] = jnp.dot(hr, w1_ref[...],
                           preferred_element_type=jnp.float32)


@functools.partial(jax.jit, static_argnames=())
def kernel(x, adj, idx, W0, W1):
    xw = pl.pallas_call(
        _matmul_small_kernel,
        out_shape=jax.ShapeDtypeStruct((N, H), jnp.float32),
    )(x, W0)

    nm = N // BM1
    h = pl.pallas_call(
        _layer1_kernel,
        grid=(nm,),
        in_specs=[
            pl.BlockSpec((BM1, N), lambda m: (m, 0)),
            pl.BlockSpec((N, H), lambda m: (0, 0)),
        ],
        out_specs=pl.BlockSpec((BM1, H), lambda m: (m, 0)),
        out_shape=jax.ShapeDtypeStruct((N, H), jnp.float32),
    )(adj, xw)

    grid_spec = pltpu.PrefetchScalarGridSpec(
        num_scalar_prefetch=1,
        grid=(NB,),
        in_specs=[
            pl.BlockSpec(memory_space=pl.ANY),
            pl.BlockSpec((N, H), lambda b, idx_ref: (0, 0)),
            pl.BlockSpec((H, C), lambda b, idx_ref: (0, 0)),
        ],
        out_specs=pl.BlockSpec((BR, C), lambda b, idx_ref: (b, 0)),
        scratch_shapes=[
            pltpu.VMEM((2, BR, N), jnp.float32),
            pltpu.SemaphoreType.DMA((2,)),
        ],
    )
    out = pl.pallas_call(
        _layer2_kernel,
        grid_spec=grid_spec,
        out_shape=jax.ShapeDtypeStruct((Q, C), jnp.float32),
    )(idx, adj, h, W1)
    return out


# speedup vs baseline: 4.2577x; 1.0244x over previous
import functools

import jax
import jax.numpy as jnp
from jax import lax
from jax.experimental import pallas as pl
from jax.experimental.pallas import tpu as pltpu

N, F, H, C, Q = 10000, 128, 16, 16, 2048

BM1 = 400
BR = 256
NB = Q // BR


def _matmul_small_kernel(a_ref, b_ref, o_ref):
    o_ref[...] = jnp.dot(a_ref[...], b_ref[...],
                         preferred_element_type=jnp.float32)


def _layer1_kernel(adj_ref, xw_ref, h_ref):
    acc = jnp.dot(adj_ref[...], xw_ref[...],
                  preferred_element_type=jnp.float32)
    h_ref[...] = jnp.maximum(acc, 0.0)


def _layer2_kernel(idx_ref, adj_hbm, h_ref, w1_ref, out_ref, buf, sem):
    b = pl.program_id(0)

    def start_batch(batch, slot):
        base = batch * BR
        for j in range(BR):
            pltpu.make_async_copy(
                adj_hbm.at[idx_ref[base + j]],
                buf.at[slot, j],
                sem.at[slot],
            ).start()

    def wait_batch(batch, slot):
        pltpu.make_async_copy(
            adj_hbm.at[pl.ds(0, BR)],
            buf.at[slot],
            sem.at[slot],
        ).wait()

    slot = lax.rem(b, 2)

    @pl.when(b == 0)
    def _():
        start_batch(0, 0)

    @pl.when(b + 1 < NB)
    def _():
        start_batch(b + 1, 1 - slot)

    wait_batch(b, slot)
    rows = buf[slot]
    hr = jnp.dot(rows, h_ref[...], preferred_element_type=jnp.float32)
    out_ref[...] = jnp.dot(hr, w1_ref[...],
                           preferred_element_type=jnp.float32)


@functools.partial(jax.jit, static_argnames=())
def kernel(x, adj, idx, W0, W1):
    xw = pl.pallas_call(
        _matmul_small_kernel,
        out_shape=jax.ShapeDtypeStruct((N, H), jnp.float32),
    )(x, W0)

    nm = N // BM1
    h = pl.pallas_call(
        _layer1_kernel,
        grid=(nm,),
        in_specs=[
            pl.BlockSpec((BM1, N), lambda m: (m, 0)),
            pl.BlockSpec((N, H), lambda m: (0, 0)),
        ],
        out_specs=pl.BlockSpec((BM1, H), lambda m: (m, 0)),
        out_shape=jax.ShapeDtypeStruct((N, H), jnp.float32),
    )(adj, xw)

    grid_spec = pltpu.PrefetchScalarGridSpec(
        num_scalar_prefetch=1,
        grid=(NB,),
        in_specs=[
            pl.BlockSpec(memory_space=pl.ANY),
            pl.BlockSpec((N, H), lambda b, idx_ref: (0, 0)),
            pl.BlockSpec((H, C), lambda b, idx_ref: (0, 0)),
        ],
        out_specs=pl.BlockSpec((BR, C), lambda b, idx_ref: (b, 0)),
        scratch_shapes=[
            pltpu.VMEM((2, BR, N), jnp.float32),
            pltpu.SemaphoreType.DMA((2,)),
        ],
    )
    out = pl.pallas_call(
        _layer2_kernel,
        grid_spec=grid_spec,
        out_shape=jax.ShapeDtypeStruct((Q, C), jnp.float32),
    )(idx, adj, h, W1)
    return out
